# Initial kernel scaffold; baseline (speedup 1.0000x reference)
import jax
import jax.numpy as jnp
from jax.experimental import pallas as pl


def kernel(x, seg, tok_table, pos_table, seg_table, gamma, beta):
    raise NotImplementedError("write your pallas kernel here")


# speedup vs baseline: 6.0392x; 6.0392x over previous
import functools

import jax
import jax.numpy as jnp
from jax import lax
from jax.experimental import pallas as pl
from jax.experimental.pallas import tpu as pltpu
from jax.experimental.pallas import tpu_sc as plsc

EPS = 1e-5

_NC = 2
_NS = 16
_NW = _NC * _NS
_CHUNK = 128


def _sc_gather_body(tok_hbm, idx_hbm, out_hbm, idx_v, rows_v, sem):
    n_tokens = idx_hbm.shape[0]
    per_w = n_tokens // _NW
    n_chunks = per_w // _CHUNK
    wid = lax.axis_index("s") * _NC + lax.axis_index("c")
    base = wid * per_w

    def step(i, carry):
        off = base + i * _CHUNK
        pltpu.sync_copy(idx_hbm.at[pl.ds(off, _CHUNK)], idx_v)
        pltpu.async_copy(tok_hbm.at[idx_v], rows_v, sem).wait()
        pltpu.sync_copy(rows_v, out_hbm.at[pl.ds(off, _CHUNK)])
        return carry

    lax.fori_loop(0, n_chunks, step, 0)


def _make_gather(n_tokens, d_model):
    mesh = plsc.VectorSubcoreMesh(core_axis_name="c", subcore_axis_name="s")
    return pl.kernel(
        _sc_gather_body,
        out_type=jax.ShapeDtypeStruct((n_tokens, d_model), jnp.float32),
        mesh=mesh,
        scratch_types=[
            pltpu.VMEM((_CHUNK,), jnp.int32),
            pltpu.VMEM((_CHUNK, d_model), jnp.float32),
            pltpu.SemaphoreType.DMA,
        ],
    )


def _ln_body(g_ref, seg_ref, pos_ref, segt_ref, gamma_ref, beta_ref, o_ref):
    e = g_ref[...]
    pos = pos_ref[...]
    st = segt_ref[...]
    sf = seg_ref[...].astype(jnp.float32)
    e = e + pos[None, :, :]
    e = e + st[0][None, None, :] + sf[..., None] * (st[1] - st[0])[None, None, :]
    mean = jnp.mean(e, axis=-1, keepdims=True)
    c = e - mean
    var = jnp.mean(c * c, axis=-1, keepdims=True)
    normed = c * lax.rsqrt(var + EPS)
    o_ref[...] = normed * gamma_ref[...][None, :] + beta_ref[...][None, :]


def kernel(x, seg, tok_table, pos_table, seg_table, gamma, beta):
    batch, seq = x.shape
    d = tok_table.shape[1]
    n_tokens = batch * seq

    x_flat = x.reshape(-1).astype(jnp.int32)
    g = _make_gather(n_tokens, d)(tok_table, x_flat)
    g3 = g.reshape(batch, seq, d)

    bb = 32
    grid = (batch // bb,)
    out = pl.pallas_call(
        _ln_body,
        grid=grid,
        in_specs=[
            pl.BlockSpec((bb, seq, d), lambda i: (i, 0, 0)),
            pl.BlockSpec((bb, seq), lambda i: (i, 0)),
            pl.BlockSpec((seq, d), lambda i: (0, 0)),
            pl.BlockSpec((2, d), lambda i: (0, 0)),
            pl.BlockSpec((1, d), lambda i: (0, 0)),
            pl.BlockSpec((1, d), lambda i: (0, 0)),
        ],
        out_specs=pl.BlockSpec((bb, seq, d), lambda i: (i, 0, 0)),
        out_shape=jax.ShapeDtypeStruct((batch, seq, d), jnp.float32),
    )(g3, seg.astype(jnp.int32), pos_table, seg_table,
      gamma.reshape(1, d), beta.reshape(1, d))
    return out
